# baseline (speedup 1.0000x reference)
import jax
import jax.numpy as jnp
from jax.experimental import pallas as pl
from jax.experimental.pallas import tpu as pltpu

_B, _S, _V, _D_EMB, _H = 64, 60, 128, 256, 1024


def _onestep_kernel(ids_ref, h0_ref, E_ref, Wx_ref, Wh_ref, b_ref, Wout_ref,
                    bout_ref, mask_ref, noise_ref,
                    ids_out_ref, h_out_ref, logits_out_ref, GX_ref):
    EWxb = (jnp.dot(E_ref[...], Wx_ref[...],
                    preferred_element_type=jnp.float32)
            + b_ref[...]).astype(jnp.bfloat16)
    iota_v = jax.lax.broadcasted_iota(jnp.int32, (_S * _B, _V), 1)
    onehot = (ids_ref[...] == iota_v).astype(jnp.bfloat16)
    GX_ref[...] = jnp.dot(onehot, EWxb,
                          preferred_element_type=jnp.float32
                          ).astype(jnp.bfloat16)

    def step(t, h):
        gx = GX_ref[pl.ds(t * _B, _B), :].astype(jnp.float32)
        gh = jnp.dot(h.astype(jnp.bfloat16), Wh_ref[...],
                     preferred_element_type=jnp.float32)
        zr = jax.nn.sigmoid(gx[:, :2 * _H] + gh[:, :2 * _H])
        z = zr[:, :_H]
        r = zr[:, _H:]
        hh = jnp.tanh(gx[:, 2 * _H:] + r * gh[:, 2 * _H:])
        return z * h + (1.0 - z) * hh

    h = jax.lax.fori_loop(0, _S, step, h0_ref[...], unroll=12)
    h_out_ref[...] = h
    logits = (jnp.dot(h, Wout_ref[...], preferred_element_type=jnp.float32)
              + bout_ref[...] + mask_ref[...])
    logits_out_ref[...] = logits
    sample = jnp.argmax(logits + noise_ref[...], axis=1).astype(jnp.int32)
    ids_out_ref[...] = sample[:, None]


def kernel(input_ids, states, E, Wx, Wh, b, Wout, bout, mask):
    ids = input_ids.astype(jnp.int32).T.reshape(_S * _B, 1)
    noise = jax.random.gumbel(jax.random.key(1), (_B, _V), jnp.float32)
    out_types = (
        jax.ShapeDtypeStruct((_B, 1), jnp.int32),
        jax.ShapeDtypeStruct((_B, _H), jnp.float32),
        jax.ShapeDtypeStruct((_B, _V), jnp.float32),
    )
    ids_out, h_out, logits = pl.pallas_call(
        _onestep_kernel,
        out_shape=out_types,
        scratch_shapes=[pltpu.VMEM((_S * _B, 3 * _H), jnp.bfloat16)],
    )(ids, states, E, Wx, Wh.astype(jnp.bfloat16), b.reshape(1, 3 * _H), Wout,
      bout.reshape(1, _V), mask.reshape(1, _V), noise)
    return ids_out.reshape(_B), h_out, logits


# speedup vs baseline: 1.0599x; 1.0599x over previous
import jax
import jax.numpy as jnp
from jax.experimental import pallas as pl
from jax.experimental.pallas import tpu as pltpu

_B, _S, _V, _D_EMB, _H = 64, 60, 128, 256, 1024


def _onestep_kernel(ids_ref, h0_ref, E_ref, Wx_ref, Wh_ref, b_ref, Wout_ref,
                    bout_ref, mask_ref, noise_ref,
                    ids_out_ref, h_out_ref, logits_out_ref, EWxb_ref):
    EWxb_ref[...] = (jnp.dot(E_ref[...], Wx_ref[...],
                             preferred_element_type=jnp.float32)
                     + b_ref[...]).astype(jnp.bfloat16)
    iota_v = jax.lax.broadcasted_iota(jnp.int32, (_V, _B), 0)

    def step(t, h):
        tok = ids_ref[pl.ds(t, 1), :]
        onehot_vb = (iota_v == tok).astype(jnp.bfloat16)
        gx = jax.lax.dot_general(onehot_vb, EWxb_ref[...],
                                 (((0,), (0,)), ((), ())),
                                 preferred_element_type=jnp.float32)
        gh = jnp.dot(h.astype(jnp.bfloat16), Wh_ref[...],
                     preferred_element_type=jnp.float32)
        zr = jax.nn.sigmoid(gx[:, :2 * _H] + gh[:, :2 * _H])
        z = zr[:, :_H]
        r = zr[:, _H:]
        hh = jnp.tanh(gx[:, 2 * _H:] + r * gh[:, 2 * _H:])
        return z * h + (1.0 - z) * hh

    h = jax.lax.fori_loop(0, _S, step, h0_ref[...], unroll=20)
    h_out_ref[...] = h
    logits = (jnp.dot(h, Wout_ref[...], preferred_element_type=jnp.float32)
              + bout_ref[...] + mask_ref[...])
    logits_out_ref[...] = logits
    sample = jnp.argmax(logits + noise_ref[...], axis=1).astype(jnp.int32)
    ids_out_ref[...] = sample[:, None]


def kernel(input_ids, states, E, Wx, Wh, b, Wout, bout, mask):
    ids = input_ids.astype(jnp.int32).T
    noise = jax.random.gumbel(jax.random.key(1), (_B, _V), jnp.float32)
    out_types = (
        jax.ShapeDtypeStruct((_B, 1), jnp.int32),
        jax.ShapeDtypeStruct((_B, _H), jnp.float32),
        jax.ShapeDtypeStruct((_B, _V), jnp.float32),
    )
    ids_out, h_out, logits = pl.pallas_call(
        _onestep_kernel,
        out_shape=out_types,
        scratch_shapes=[pltpu.VMEM((_V, 3 * _H), jnp.bfloat16)],
    )(ids, states, E, Wx, Wh.astype(jnp.bfloat16), b.reshape(1, 3 * _H), Wout,
      bout.reshape(1, _V), mask.reshape(1, _V), noise)
    return ids_out.reshape(_B), h_out, logits


# speedup vs baseline: 1.0629x; 1.0028x over previous
import jax
import jax.numpy as jnp
from jax.experimental import pallas as pl
from jax.experimental.pallas import tpu as pltpu

_B, _S, _V, _D_EMB, _H = 64, 60, 128, 256, 1024


def _onestep_kernel(ids_ref, h0_ref, E_ref, Wx_ref, Wh_ref, b_ref, Wout_ref,
                    bout_ref, mask_ref, noise_ref,
                    ids_out_ref, h_out_ref, logits_out_ref, EWxb_ref):
    EWxb_ref[...] = (jnp.dot(E_ref[...], Wx_ref[...],
                             preferred_element_type=jnp.float32)
                     + b_ref[...]).astype(jnp.bfloat16)
    iota_v = jax.lax.broadcasted_iota(jnp.int32, (_V, _B), 0)

    def step(t, h):
        tok = ids_ref[pl.ds(t, 1), :]
        onehot_vb = (iota_v == tok).astype(jnp.bfloat16)
        gx = jax.lax.dot_general(onehot_vb, EWxb_ref[...],
                                 (((0,), (0,)), ((), ())),
                                 preferred_element_type=jnp.float32)
        gh = jnp.dot(h.astype(jnp.bfloat16), Wh_ref[...],
                     preferred_element_type=jnp.float32)
        zr = jax.nn.sigmoid(gx[:, :2 * _H] + gh[:, :2 * _H])
        z = zr[:, :_H]
        r = zr[:, _H:]
        hh = jnp.tanh(gx[:, 2 * _H:] + r * gh[:, 2 * _H:])
        return z * h + (1.0 - z) * hh

    h = jax.lax.fori_loop(0, _S, step, h0_ref[...], unroll=30)
    h_out_ref[...] = h
    logits = (jnp.dot(h, Wout_ref[...], preferred_element_type=jnp.float32)
              + bout_ref[...] + mask_ref[...])
    logits_out_ref[...] = logits
    sample = jnp.argmax(logits + noise_ref[...], axis=1).astype(jnp.int32)
    ids_out_ref[...] = sample[:, None]


def kernel(input_ids, states, E, Wx, Wh, b, Wout, bout, mask):
    ids = input_ids.astype(jnp.int32).T
    noise = jax.random.gumbel(jax.random.key(1), (_B, _V), jnp.float32)
    out_types = (
        jax.ShapeDtypeStruct((_B, 1), jnp.int32),
        jax.ShapeDtypeStruct((_B, _H), jnp.float32),
        jax.ShapeDtypeStruct((_B, _V), jnp.float32),
    )
    ids_out, h_out, logits = pl.pallas_call(
        _onestep_kernel,
        out_shape=out_types,
        scratch_shapes=[pltpu.VMEM((_V, 3 * _H), jnp.bfloat16)],
    )(ids, states, E, Wx, Wh.astype(jnp.bfloat16), b.reshape(1, 3 * _H), Wout,
      bout.reshape(1, _V), mask.reshape(1, _V), noise)
    return ids_out.reshape(_B), h_out, logits
